# baseline (speedup 1.0000x reference)
import jax
import jax.numpy as jnp
from jax.experimental import pallas as pl
from jax.experimental.pallas import tpu as pltpu

_TN = 4096
_CH = 256
_TC = 512


def _chamfer_kernel(a_ref, bt_ref, o_ref, cmin_ref, acc_ref, id_ref):
    bidx = pl.program_id(0)
    r = pl.program_id(1)
    nb = pl.num_programs(0)
    nr = pl.num_programs(1)

    @pl.when(jnp.logical_and(bidx == 0, r == 0))
    def _init_acc():
        acc_ref[0] = jnp.float32(0.0)
        rows = jax.lax.broadcasted_iota(jnp.int32, (_TC, _TC), 0)
        cols = jax.lax.broadcasted_iota(jnp.int32, (_TC, _TC), 1)
        id_ref[...] = (rows == cols).astype(jnp.bfloat16)

    @pl.when(r == 0)
    def _init_cmin():
        cmin_ref[...] = jnp.full_like(cmin_ref, jnp.inf)

    m_total = bt_ref.shape[1]
    n_total = nr * _TN
    inv_n = 1.0 / (n_total * nb)

    ident = id_ref[...]
    parts = []
    for c in range(m_total // _TC):
        bt_c = bt_ref[0, c * _TC : (c + 1) * _TC, :]
        parts.append(
            jax.lax.dot_general(
                bt_c,
                ident,
                (((0,), (0,)), ((), ())),
                preferred_element_type=jnp.float32,
            )
        )
    bm = jnp.concatenate(parts, axis=1).astype(jnp.bfloat16)

    for c in range(_TN // _CH):
        a_c = a_ref[0, c * _CH : (c + 1) * _CH, :]
        d = jax.lax.dot_general(
            a_c,
            bm,
            (((1,), (0,)), ((), ())),
            preferred_element_type=jnp.float32,
        )
        acc_ref[0] += jnp.sum(jnp.min(d, axis=1)) * inv_n
        cmin_ref[...] = jnp.minimum(
            cmin_ref[...], jnp.min(d, axis=0, keepdims=True)
        )

    @pl.when(r == nr - 1)
    def _finish_batch():
        acc_ref[0] += jnp.sum(cmin_ref[...]) / (m_total * nb)

    @pl.when(jnp.logical_and(bidx == nb - 1, r == nr - 1))
    def _write_out():
        o_ref[...] = jnp.full_like(o_ref, acc_ref[0])


def _split_bf16(v):
    u = jax.lax.bitcast_convert_type(v, jnp.uint32)
    hi = jax.lax.bitcast_convert_type(u & jnp.uint32(0xFFFF0000), jnp.float32)
    lo = v - hi
    return hi.astype(jnp.bfloat16), lo.astype(jnp.bfloat16)


def kernel(x, y):
    B, N, _ = x.shape
    M = y.shape[1]

    s = -2.0 * x
    sh, sl = _split_bf16(s)
    yh, yl = _split_bf16(y)
    xx = jnp.sum(x * x, axis=-1, keepdims=True)
    yy = jnp.sum(y * y, axis=-1, keepdims=True)
    xxh, xxl = _split_bf16(xx)
    yyh, yyl = _split_bf16(yy)
    ones_n = jnp.ones((B, N, 1), jnp.bfloat16)
    ones_m = jnp.ones((B, M, 1), jnp.bfloat16)

    a = jnp.concatenate(
        [sh, sh, sl, sl, xxh, xxl, ones_n, ones_n], axis=-1
    )
    bm_t = jnp.concatenate(
        [yh, yl, yh, yl, ones_m, ones_m, yyh, yyl], axis=-1
    )

    out = pl.pallas_call(
        _chamfer_kernel,
        grid=(B, N // _TN),
        in_specs=[
            pl.BlockSpec((1, _TN, 16), lambda b, r: (b, r, 0)),
            pl.BlockSpec((1, M, 16), lambda b, r: (b, 0, 0)),
        ],
        out_specs=pl.BlockSpec((1, 1), lambda b, r: (0, 0)),
        out_shape=jax.ShapeDtypeStruct((1, 1), jnp.float32),
        scratch_shapes=[
            pltpu.VMEM((1, M), jnp.float32),
            pltpu.SMEM((1,), jnp.float32),
            pltpu.VMEM((_TC, _TC), jnp.bfloat16),
        ],
    )(a, bm_t)
    return out[0, 0]


# speedup vs baseline: 3.5384x; 3.5384x over previous
import jax
import jax.numpy as jnp
from jax.experimental import pallas as pl
from jax.experimental.pallas import tpu as pltpu

_TN = 4096
_CH = 256


def _chamfer_kernel(a_ref, b8_ref, o_ref, cmin_ref, acc_ref):
    bidx = pl.program_id(0)
    r = pl.program_id(1)
    nb = pl.num_programs(0)
    nr = pl.num_programs(1)

    @pl.when(jnp.logical_and(bidx == 0, r == 0))
    def _init_acc():
        acc_ref[0] = jnp.float32(0.0)

    @pl.when(r == 0)
    def _init_cmin():
        cmin_ref[...] = jnp.full_like(cmin_ref, jnp.inf)

    m_total = b8_ref.shape[2]
    n_total = nr * _TN
    inv_n = 1.0 / (n_total * nb)

    p = b8_ref[0].astype(jnp.float32)
    ones_m = jnp.ones((1, m_total), jnp.float32)
    bm = jnp.concatenate(
        [p[0:3], p[3:6], p[0:3], p[3:6], ones_m, ones_m, p[6:7], p[7:8]],
        axis=0,
    ).astype(jnp.bfloat16)

    for c in range(_TN // _CH):
        a_c = a_ref[0, c * _CH : (c + 1) * _CH, :]
        d = jax.lax.dot_general(
            a_c,
            bm,
            (((1,), (0,)), ((), ())),
            preferred_element_type=jnp.float32,
        )
        acc_ref[0] += jnp.sum(jnp.min(d, axis=1)) * inv_n
        cmin_ref[...] = jnp.minimum(
            cmin_ref[...], jnp.min(d, axis=0, keepdims=True)
        )

    @pl.when(r == nr - 1)
    def _finish_batch():
        acc_ref[0] += jnp.sum(cmin_ref[...]) / (m_total * nb)

    @pl.when(jnp.logical_and(bidx == nb - 1, r == nr - 1))
    def _write_out():
        o_ref[...] = jnp.full_like(o_ref, acc_ref[0])


def _split_bf16(v):
    u = jax.lax.bitcast_convert_type(v, jnp.uint32)
    hi = jax.lax.bitcast_convert_type(u & jnp.uint32(0xFFFF0000), jnp.float32)
    lo = v - hi
    return hi.astype(jnp.bfloat16), lo.astype(jnp.bfloat16)


def kernel(x, y):
    B, N, _ = x.shape
    M = y.shape[1]

    s = -2.0 * x
    sh, sl = _split_bf16(s)
    yh, yl = _split_bf16(y)
    xx = jnp.sum(x * x, axis=-1, keepdims=True)
    yy = jnp.sum(y * y, axis=-1, keepdims=True)
    xxh, xxl = _split_bf16(xx)
    yyh, yyl = _split_bf16(yy)
    ones_n = jnp.ones((B, N, 1), jnp.bfloat16)

    a = jnp.concatenate(
        [sh, sh, sl, sl, xxh, xxl, ones_n, ones_n], axis=-1
    )
    b8_t = jnp.concatenate([yh, yl, yyh, yyl], axis=-1)
    b8 = jnp.swapaxes(b8_t, 1, 2)

    out = pl.pallas_call(
        _chamfer_kernel,
        grid=(B, N // _TN),
        in_specs=[
            pl.BlockSpec((1, _TN, 16), lambda b, r: (b, r, 0)),
            pl.BlockSpec((1, 8, M), lambda b, r: (b, 0, 0)),
        ],
        out_specs=pl.BlockSpec((1, 1), lambda b, r: (0, 0)),
        out_shape=jax.ShapeDtypeStruct((1, 1), jnp.float32),
        scratch_shapes=[
            pltpu.VMEM((1, M), jnp.float32),
            pltpu.SMEM((1,), jnp.float32),
        ],
    )(a, b8)
    return out[0, 0]
